# Initial kernel scaffold; baseline (speedup 1.0000x reference)
import jax
import jax.numpy as jnp
from jax.experimental import pallas as pl


def kernel(seq_len, emb_weight):
    raise NotImplementedError("write your pallas kernel here")


# speedup vs baseline: 6.3811x; 6.3811x over previous
import jax
import jax.numpy as jnp
from jax import lax
from jax.experimental import pallas as pl
from jax.experimental.pallas import tpu as pltpu
from jax.experimental.pallas import tpu_sc as plsc

MAX_LEN = 2048
D_MODEL = 64
S = MAX_LEN
TFLAT = (2 * MAX_LEN - 1) * D_MODEL
ROW = S * D_MODEL
NUM_WORKERS = 32
ROWS_PER_W = S // NUM_WORKERS
NCHUNK = 2
COLW = ROW // NCHUNK
SPAN = COLW + (ROWS_PER_W - 1) * D_MODEL
FIRE = 16


def _sc_body(tflat_hbm, out_hbm, stage, sem):
    c = lax.axis_index("c")
    s = lax.axis_index("s")
    wid = s * 2 + c
    base = wid * ROWS_PER_W
    for jc in range(NCHUNK):
        lo = (S - ROWS_PER_W - base) * D_MODEL + jc * COLW
        pltpu.sync_copy(tflat_hbm.at[pl.ds(lo, SPAN)], stage)
        for g in range(0, ROWS_PER_W, FIRE):
            copies = []
            for r in range(g, g + FIRE):
                src_off = (ROWS_PER_W - 1 - r) * D_MODEL
                dst = (base + r) * ROW + jc * COLW
                copies.append(
                    pltpu.async_copy(
                        stage.at[pl.ds(src_off, COLW)],
                        out_hbm.at[pl.ds(dst, COLW)],
                        sem,
                    )
                )
            for cp in copies:
                cp.wait()


def kernel(seq_len, emb_weight):
    tflat = emb_weight.reshape(TFLAT)
    mesh = plsc.VectorSubcoreMesh(core_axis_name="c", subcore_axis_name="s")
    out = pl.kernel(
        _sc_body,
        out_type=jax.ShapeDtypeStruct((S * ROW,), jnp.float32),
        mesh=mesh,
        scratch_types=[
            pltpu.VMEM((SPAN,), jnp.float32),
            pltpu.SemaphoreType.DMA,
        ],
    )(tflat)
    return out.reshape(S, S, D_MODEL)


# speedup vs baseline: 29.6412x; 4.6451x over previous
import jax
import jax.numpy as jnp
from jax.experimental import pallas as pl

MAX_LEN = 2048
D_MODEL = 64
S = MAX_LEN
GB = 128
NG = S // GB
JB = 128
NJ = S // JB


def _tc_body(tab_ref, out_ref):
    g = pl.program_id(0)
    jb = pl.program_id(1)
    col0 = pl.multiple_of((NG - 1 - g + jb) * 128, 128)
    x = tab_ref[:, pl.ds(col0, 2 * JB)]
    for k in range(GB):
        out_ref[k, :, :] = x[:, 127 - k : 255 - k]


def kernel(seq_len, emb_weight):
    tab_t = jnp.pad(emb_weight.T, ((0, 0), (0, 1)))
    out = pl.pallas_call(
        _tc_body,
        grid=(NG, NJ),
        in_specs=[pl.BlockSpec((D_MODEL, 2 * S), lambda g, j: (0, 0))],
        out_specs=pl.BlockSpec((GB, D_MODEL, JB), lambda g, j: (g, 0, j)),
        out_shape=jax.ShapeDtypeStruct((S, D_MODEL, S), jnp.float32),
    )(tab_t)
    return out.transpose(0, 2, 1)


# speedup vs baseline: 42.2891x; 1.4267x over previous
import jax
import jax.numpy as jnp
from jax import lax
from jax.experimental import pallas as pl
from jax.experimental.pallas import tpu as pltpu
from jax.experimental.pallas import tpu_sc as plsc

MAX_LEN = 2048
D_MODEL = 64
S = MAX_LEN
NUM_WORKERS = 32
ROWS_PER_W = S // NUM_WORKERS
MGRP = 32
RGRP = 32
WIN = 8 * (MGRP - 1) + S + 8


def _sc_body(tab_hbm, out_hbm, stage, sem):
    c = lax.axis_index("c")
    sub = lax.axis_index("s")
    wid = sub * 2 + c
    a = wid % 8
    p = wid // 8
    ca = (7 - a) % 8
    for g in range(ROWS_PER_W // MGRP):
        q0 = 64 * p + MGRP * g
        smax = (S - 1) - a - 8 * q0
        smin = smax - 8 * (MGRP - 1)
        w0 = pl.multiple_of(smin - ca, 8)
        for rg in range(D_MODEL // RGRP):
            pltpu.sync_copy(
                tab_hbm.at[a, pl.ds(RGRP * rg, RGRP), pl.ds(w0, WIN)], stage
            )

            def body(k, _):
                i = a + 8 * (q0 + k)
                off = 8 * (MGRP - 1 - k)
                copies = []
                for dbl in range(RGRP // 8):
                    for jb in range(S // 128):
                        copies.append(
                            pltpu.async_copy(
                                stage.at[pl.ds(8 * dbl, 8),
                                         pl.ds(off + 128 * jb, 128)],
                                out_hbm.at[i, (RGRP // 8) * rg + dbl, jb],
                                sem,
                            )
                        )
                for cp in copies:
                    cp.wait()
                return 0

            lax.fori_loop(0, MGRP, body, 0)


def kernel(seq_len, emb_weight):
    tab_t = jnp.pad(emb_weight.T, ((0, 0), (0, 9)))
    planes = [tab_t[:, (7 - A) % 8 : (7 - A) % 8 + 4096] for A in range(8)]
    tab8 = jnp.stack(planes)
    mesh = plsc.VectorSubcoreMesh(core_axis_name="c", subcore_axis_name="s")
    out = pl.kernel(
        _sc_body,
        out_type=jax.ShapeDtypeStruct((S, 8, 16, 8, 128), jnp.float32),
        mesh=mesh,
        scratch_types=[
            pltpu.VMEM((RGRP, WIN), jnp.float32),
            pltpu.SemaphoreType.DMA,
        ],
        compiler_params=pltpu.CompilerParams(use_tc_tiling_on_sc=False),
    )(tab8)
    p = out.transpose(0, 1, 3, 2, 4).reshape(S, D_MODEL, S)
    return p.transpose(0, 2, 1)


# speedup vs baseline: 43.8420x; 1.0367x over previous
import jax
import jax.numpy as jnp
from jax import lax
from jax.experimental import pallas as pl
from jax.experimental.pallas import tpu as pltpu
from jax.experimental.pallas import tpu_sc as plsc

MAX_LEN = 2048
D_MODEL = 64
S = MAX_LEN
NUM_WORKERS = 32
ROWS_PER_W = S // NUM_WORKERS
MGRP = 64
RGRP = 32
WIN = 8 * (MGRP - 1) + S + 8


def _sc_body(tab_hbm, out_hbm, stage, sem):
    c = lax.axis_index("c")
    sub = lax.axis_index("s")
    wid = sub * 2 + c
    a = wid % 8
    p = wid // 8
    ca = (7 - a) % 8
    for g in range(ROWS_PER_W // MGRP):
        q0 = 64 * p + MGRP * g
        smax = (S - 1) - a - 8 * q0
        smin = smax - 8 * (MGRP - 1)
        w0 = pl.multiple_of(smin - ca, 8)
        for rg in range(D_MODEL // RGRP):
            pltpu.sync_copy(
                tab_hbm.at[a, pl.ds(RGRP * rg, RGRP), pl.ds(w0, WIN)], stage
            )

            def body(k, _):
                i = a + 8 * (q0 + k)
                off = 8 * (MGRP - 1 - k)
                copies = []
                for dbl in range(RGRP // 8):
                    for jb in range(S // 128):
                        copies.append(
                            pltpu.async_copy(
                                stage.at[pl.ds(8 * dbl, 8),
                                         pl.ds(off + 128 * jb, 128)],
                                out_hbm.at[i, (RGRP // 8) * rg + dbl, jb],
                                sem,
                            )
                        )
                for cp in copies:
                    cp.wait()
                return 0

            lax.fori_loop(0, MGRP, body, 0)


def kernel(seq_len, emb_weight):
    tab_t = jnp.pad(emb_weight.T, ((0, 0), (0, 9)))
    planes = [tab_t[:, (7 - A) % 8 : (7 - A) % 8 + 4096] for A in range(8)]
    tab8 = jnp.stack(planes)
    mesh = plsc.VectorSubcoreMesh(core_axis_name="c", subcore_axis_name="s")
    out = pl.kernel(
        _sc_body,
        out_type=jax.ShapeDtypeStruct((S, 8, 16, 8, 128), jnp.float32),
        mesh=mesh,
        scratch_types=[
            pltpu.VMEM((RGRP, WIN), jnp.float32),
            pltpu.SemaphoreType.DMA,
        ],
        compiler_params=pltpu.CompilerParams(use_tc_tiling_on_sc=False),
    )(tab8)
    p = out.transpose(0, 1, 3, 2, 4).reshape(S, D_MODEL, S)
    return p.transpose(0, 2, 1)
